# baseline (speedup 1.0000x reference)
import functools
import math

import jax
import jax.numpy as jnp
from jax.experimental import pallas as pl
from jax.experimental.pallas import tpu as pltpu

_VMEM_LIMIT_BYTES = 48 << 20
_TABLE_DMA_CHUNKS = 8
_WRITE_SLOTS = 8


def _round_up(x: int, m: int) -> int:
    return (x + m - 1) // m * m


def _gather_kernel(ids_ref, emb_hbm, out_hbm, tbl, gtile, wbufs,
                   tsems, wsems, *, tb, nb, half, n_chunks,
                   rows_per_chunk, scale):
    h = pl.program_id(0)
    blk = pl.program_id(1)
    w_slots = wbufs.shape[0]

    @pl.when(blk == 0)
    def _load_half_table():
        col = pl.multiple_of(h * half, half)
        for c in range(n_chunks):
            pltpu.make_async_copy(
                emb_hbm.at[pl.ds(c * rows_per_chunk, rows_per_chunk), :,
                           pl.ds(col, half)],
                tbl.at[pl.ds(c * rows_per_chunk, rows_per_chunk)],
                tsems.at[c],
            ).start()
        for c in range(n_chunks):
            pltpu.make_async_copy(
                emb_hbm.at[pl.ds(c * rows_per_chunk, rows_per_chunk), :,
                           pl.ds(col, half)],
                tbl.at[pl.ds(c * rows_per_chunk, rows_per_chunk)],
                tsems.at[c],
            ).wait()

    base = blk * tb
    for mi in range(tb):
        tok = ids_ref[base + mi]
        gtile[mi, 0] = tbl[tok, 0]

    w = jax.lax.rem(blk, w_slots)
    col = pl.multiple_of(h * half, half)

    def _write_copy(slot, block):
        return pltpu.make_async_copy(
            wbufs.at[slot],
            out_hbm.at[pl.ds(block * tb, tb), pl.ds(col, half)],
            wsems.at[slot],
        )

    @pl.when(blk >= w_slots)
    def _reclaim():
        _write_copy(w, blk - w_slots).wait()

    wbufs[w] = gtile[:, 0, :] * scale
    _write_copy(w, blk).start()

    @pl.when(blk == nb - 1)
    def _drain():
        for j in range(w_slots):
            _write_copy(j, blk).wait()


def kernel(tokens: jax.Array, emb_table: jax.Array) -> jax.Array:
    seq_len, batch = tokens.shape
    vocab, emb = emb_table.shape
    n = seq_len * batch
    scale = math.sqrt(emb)

    emb_p = _round_up(emb, 256)
    if emb_p != emb:
        emb_table = jnp.pad(emb_table, ((0, 0), (0, emb_p - emb)))
    half = emb_p // 2

    ids = jnp.clip(tokens.reshape(n).astype(jnp.int32), 0, vocab - 1)

    tb = 512
    n_pad = _round_up(n, tb)
    if n_pad != n:
        ids = jnp.pad(ids, (0, n_pad - n))
    nb = n_pad // tb

    n_chunks = _TABLE_DMA_CHUNKS
    while vocab % n_chunks != 0:
        n_chunks //= 2
    rows_per_chunk = vocab // n_chunks

    w_slots = min(_WRITE_SLOTS, nb)

    emb3 = emb_table.reshape(vocab, 1, emb_p)

    grid_spec = pltpu.PrefetchScalarGridSpec(
        num_scalar_prefetch=1,
        grid=(2, nb),
        in_specs=[pl.BlockSpec(memory_space=pl.ANY)],
        out_specs=pl.BlockSpec(memory_space=pl.ANY),
        scratch_shapes=[
            pltpu.VMEM((vocab, 1, half), emb_table.dtype),
            pltpu.VMEM((tb, 1, half), emb_table.dtype),
            pltpu.VMEM((w_slots, tb, half), emb_table.dtype),
            pltpu.SemaphoreType.DMA((n_chunks,)),
            pltpu.SemaphoreType.DMA((w_slots,)),
        ],
    )
    out = pl.pallas_call(
        functools.partial(_gather_kernel, tb=tb, nb=nb, half=half,
                          n_chunks=n_chunks, rows_per_chunk=rows_per_chunk,
                          scale=scale),
        out_shape=jax.ShapeDtypeStruct((n_pad, emb_p), emb_table.dtype),
        grid_spec=grid_spec,
        compiler_params=pltpu.CompilerParams(
            dimension_semantics=("parallel", "arbitrary"),
            vmem_limit_bytes=_VMEM_LIMIT_BYTES,
        ),
    )(ids, emb3)

    return out[:n, :emb].reshape(seq_len, batch, emb)


# speedup vs baseline: 1.3038x; 1.3038x over previous
import functools
import math

import jax
import jax.numpy as jnp
from jax.experimental import pallas as pl
from jax.experimental.pallas import tpu as pltpu

_VMEM_LIMIT_BYTES = 48 << 20
_SLOTS = 6
_LOOKAHEAD = 2


def _round_up(x: int, m: int) -> int:
    return (x + m - 1) // m * m


def _gather_kernel(ids_ref, emb_hbm, out_hbm, rbufs, rsems, wsems, *,
                   tb, nbc, scale):
    h = pl.program_id(0)
    blk = pl.program_id(1)
    s = rbufs.shape[0]
    lookahead = _LOOKAHEAD

    def issue_block(b, slot):
        tok_base = (h * nbc + b) * tb
        for mi in range(tb):
            tok = ids_ref[tok_base + mi]
            pltpu.make_async_copy(
                emb_hbm.at[pl.ds(tok, 1)],
                rbufs.at[slot].at[pl.ds(mi, 1)],
                rsems.at[slot],
            ).start()

    def wait_block(slot):
        for mi in range(tb):
            pltpu.make_async_copy(
                emb_hbm.at[pl.ds(0, 1)],
                rbufs.at[slot].at[pl.ds(mi, 1)],
                rsems.at[slot],
            ).wait()

    def write_copy(slot, b):
        return pltpu.make_async_copy(
            rbufs.at[slot],
            out_hbm.at[pl.ds((h * nbc + b) * tb, tb)],
            wsems.at[slot],
        )

    @pl.when(blk == 0)
    def _warmup():
        for g in range(min(lookahead, nbc)):
            issue_block(g, g % s)

    g = blk + lookahead
    if lookahead < s:
        @pl.when(g < nbc)
        def _issue_ahead():
            gs = jax.lax.rem(g, s)

            @pl.when(g >= s)
            def _reclaim():
                write_copy(gs, g - s).wait()

            issue_block(g, gs)

    slot = jax.lax.rem(blk, s)
    wait_block(slot)
    rbufs[slot] = rbufs[slot] * scale
    write_copy(slot, blk).start()

    @pl.when(blk == nbc - 1)
    def _drain():
        for b in range(max(0, nbc - s), nbc):
            write_copy(b % s, b).wait()


def kernel(tokens: jax.Array, emb_table: jax.Array) -> jax.Array:
    seq_len, batch = tokens.shape
    vocab, emb = emb_table.shape
    n = seq_len * batch
    scale = math.sqrt(emb)

    emb_p = _round_up(emb, 128)
    if emb_p != emb:
        emb_table = jnp.pad(emb_table, ((0, 0), (0, emb_p - emb)))

    ids = jnp.clip(tokens.reshape(n).astype(jnp.int32), 0, vocab - 1)

    tb = 512
    n_pad = _round_up(n, 2 * tb)
    if n_pad != n:
        ids = jnp.pad(ids, (0, n_pad - n))
    nbc = n_pad // tb // 2

    grid_spec = pltpu.PrefetchScalarGridSpec(
        num_scalar_prefetch=1,
        grid=(2, nbc),
        in_specs=[pl.BlockSpec(memory_space=pl.ANY)],
        out_specs=pl.BlockSpec(memory_space=pl.ANY),
        scratch_shapes=[
            pltpu.VMEM((_SLOTS, tb, emb_p), emb_table.dtype),
            pltpu.SemaphoreType.DMA((_SLOTS,)),
            pltpu.SemaphoreType.DMA((_SLOTS,)),
        ],
    )
    out = pl.pallas_call(
        functools.partial(_gather_kernel, tb=tb, nbc=nbc, scale=scale),
        out_shape=jax.ShapeDtypeStruct((n_pad, emb_p), emb_table.dtype),
        grid_spec=grid_spec,
        compiler_params=pltpu.CompilerParams(
            dimension_semantics=("parallel", "arbitrary"),
            vmem_limit_bytes=_VMEM_LIMIT_BYTES,
        ),
    )(ids, emb_table)

    return out[:n, :emb].reshape(seq_len, batch, emb)
